# Initial kernel scaffold; baseline (speedup 1.0000x reference)
import jax
import jax.numpy as jnp
from jax.experimental import pallas as pl


def kernel(x, edge_index, gc1_w1, gc1_b1, gc1_w2, gc1_b2, gc2_w1, gc2_b1, gc2_w2, gc2_b2, gc3_w1, gc3_b1, gc3_w2, gc3_b2, lin1_w, lin1_b, lin2_w, lin2_b):
    raise NotImplementedError("write your pallas kernel here")


# speedup vs baseline: 2.8779x; 2.8779x over previous
import functools

import jax
import jax.numpy as jnp
from jax import lax
from jax.experimental import pallas as pl
from jax.experimental.pallas import tpu as pltpu
from jax.experimental.pallas import tpu_sc as plsc

N = 10000
F = 128
E = 320000
NCLASS = 40

NC = 2
NS = 16
NW = NC * NS

CHUNK = 128
CPW = 80
EPW = CPW * CHUNK
EPAD = NW * EPW
ZROWS = 10112
RPS = ZROWS // NS

BLK = 1000

_P = lax.Precision.HIGHEST


def _sc_agg(h, src_w, dst_w, zeros_rows):
    mesh = plsc.VectorSubcoreMesh(core_axis_name="c", subcore_axis_name="s")

    @functools.partial(
        pl.kernel,
        out_type=jax.ShapeDtypeStruct((NC, ZROWS, F), jnp.float32),
        mesh=mesh,
        scratch_types=[
            pltpu.VMEM((CPW, CHUNK), jnp.int32),
            pltpu.VMEM((CPW, CHUNK), jnp.int32),
            pltpu.VMEM((CHUNK, F), jnp.float32),
            pltpu.VMEM_SHARED((ZROWS, F), jnp.float32),
            pltpu.SemaphoreType.DMA,
        ],
    )
    def k(h_hbm, src_hbm, dst_hbm, z_hbm, out_hbm, sidx, didx, buf, acc, sem):
        c = lax.axis_index("c")
        s = lax.axis_index("s")
        w = c * NS + s
        pltpu.sync_copy(z_hbm, acc.at[pl.ds(s * RPS, RPS)])
        pltpu.sync_copy(src_hbm.at[w], sidx)
        pltpu.sync_copy(dst_hbm.at[w], didx)
        plsc.subcore_barrier()

        @pl.loop(0, CPW)
        def _(g):
            pltpu.async_copy(h_hbm.at[sidx.at[g]], buf, sem).wait()
            pltpu.sync_copy(buf, acc.at[didx.at[g]], add=True)

        plsc.subcore_barrier()
        pltpu.sync_copy(acc.at[pl.ds(s * RPS, RPS)],
                        out_hbm.at[c].at[pl.ds(s * RPS, RPS)])

    return k(h, src_w, dst_w, zeros_rows)


def _tc_mlp(x, parts, w1, b1, w2, b2):
    def body(x_ref, p_ref, w1_ref, b1_ref, w2_ref, b2_ref, o_ref):
        h = x_ref[...] + p_ref[0] + p_ref[1]
        a = jnp.dot(h, w1_ref[...], precision=_P,
                    preferred_element_type=jnp.float32) + b1_ref[...]
        a = jnp.maximum(a, 0.0)
        o = jnp.dot(a, w2_ref[...], precision=_P,
                    preferred_element_type=jnp.float32) + b2_ref[...]
        o_ref[...] = jnp.maximum(o, 0.0)

    return pl.pallas_call(
        body,
        grid=(N // BLK,),
        in_specs=[
            pl.BlockSpec((BLK, F), lambda i: (i, 0)),
            pl.BlockSpec((NC, BLK, F), lambda i: (0, i, 0)),
            pl.BlockSpec((F, F), lambda i: (0, 0)),
            pl.BlockSpec((1, F), lambda i: (0, 0)),
            pl.BlockSpec((F, F), lambda i: (0, 0)),
            pl.BlockSpec((1, F), lambda i: (0, 0)),
        ],
        out_specs=pl.BlockSpec((BLK, F), lambda i: (i, 0)),
        out_shape=jax.ShapeDtypeStruct((N, F), jnp.float32),
    )(x, parts, w1, b1.reshape(1, F), w2, b2.reshape(1, F))


def _tc_head(h1, h2, h3, lw1, lb1, lw2p, lb2p):
    def body(h1_ref, h2_ref, h3_ref, w1_ref, b1_ref, w2_ref, b2_ref, o_ref):
        t = (jnp.dot(h1_ref[...], w1_ref[0], precision=_P,
                     preferred_element_type=jnp.float32)
             + jnp.dot(h2_ref[...], w1_ref[1], precision=_P,
                       preferred_element_type=jnp.float32)
             + jnp.dot(h3_ref[...], w1_ref[2], precision=_P,
                       preferred_element_type=jnp.float32)) + b1_ref[...]
        t = jnp.maximum(t, 0.0)
        o = jnp.dot(t, w2_ref[...], precision=_P,
                    preferred_element_type=jnp.float32) + b2_ref[...]
        m = jnp.max(o, axis=1, keepdims=True)
        lse = jnp.log(jnp.sum(jnp.exp(o - m), axis=1, keepdims=True)) + m
        o_ref[...] = o - lse

    return pl.pallas_call(
        body,
        grid=(N // BLK,),
        in_specs=[
            pl.BlockSpec((BLK, F), lambda i: (i, 0)),
            pl.BlockSpec((BLK, F), lambda i: (i, 0)),
            pl.BlockSpec((BLK, F), lambda i: (i, 0)),
            pl.BlockSpec((3, F, 3 * F), lambda i: (0, 0, 0)),
            pl.BlockSpec((1, 3 * F), lambda i: (0, 0)),
            pl.BlockSpec((3 * F, F), lambda i: (0, 0)),
            pl.BlockSpec((1, F), lambda i: (0, 0)),
        ],
        out_specs=pl.BlockSpec((BLK, F), lambda i: (i, 0)),
        out_shape=jax.ShapeDtypeStruct((N, F), jnp.float32),
    )(h1, h2, h3, lw1, lb1.reshape(1, 3 * F), lw2p, lb2p.reshape(1, F))


def kernel(x, edge_index, gc1_w1, gc1_b1, gc1_w2, gc1_b2, gc2_w1, gc2_b1,
           gc2_w2, gc2_b2, gc3_w1, gc3_b1, gc3_w2, gc3_b2, lin1_w, lin1_b,
           lin2_w, lin2_b):
    src = edge_index[0]
    dst = edge_index[1]
    pad = EPAD - E
    src_w = jnp.concatenate(
        [src, jnp.zeros((pad,), src.dtype)]).reshape(NW, CPW, CHUNK)
    dst_w = jnp.concatenate(
        [dst, jnp.full((pad,), N, dst.dtype)]).reshape(NW, CPW, CHUNK)
    zeros_rows = jnp.zeros((RPS, F), jnp.float32)

    p1 = _sc_agg(x, src_w, dst_w, zeros_rows)
    h1 = _tc_mlp(x, p1, gc1_w1, gc1_b1, gc1_w2, gc1_b2)
    p2 = _sc_agg(h1, src_w, dst_w, zeros_rows)
    h2 = _tc_mlp(h1, p2, gc2_w1, gc2_b1, gc2_w2, gc2_b2)
    p3 = _sc_agg(h2, src_w, dst_w, zeros_rows)
    h3 = _tc_mlp(h2, p3, gc3_w1, gc3_b1, gc3_w2, gc3_b2)

    lw1 = lin1_w.reshape(3, F, 3 * F)
    lw2p = jnp.pad(lin2_w, ((0, 0), (0, F - NCLASS)))
    lb2p = jnp.concatenate(
        [lin2_b, jnp.full((F - NCLASS,), -1e30, jnp.float32)])
    out = _tc_head(h1, h2, h3, lw1, lin1_b, lw2p, lb2p)
    return out[:, :NCLASS]
